# baseline (speedup 1.0000x reference)
import jax
import jax.numpy as jnp
from jax import lax
from jax.experimental import pallas as pl
from jax.experimental.pallas import tpu as pltpu
from jax.experimental.pallas import tpu_sc as plsc

N = 100000
F = 128
G = 64
EPS = 1e-05
BLK = 10000
NBLK = N // BLK
HEAD_BLKS = 8
NW = 32
TW = 640
TAIL = NW * TW
HEAD = N - TAIL
NF16 = F // 16


def _sc_stats(x_hbm, b_hbm, psum_hbm, pcnt_hbm, xbuf, ids, acc, cnt):
    wid = lax.axis_index("s") * 2 + lax.axis_index("c")
    start = HEAD + wid * TW
    def _zrow(g, _):
        for f in range(2 * NF16):
            acc[g, pl.ds(f * 16, 16)] = jnp.zeros((16,), jnp.float32)
        cnt[g, :] = jnp.zeros((16,), jnp.float32)
        return _
    lax.fori_loop(0, G, _zrow, 0)

    pltpu.sync_copy(x_hbm.at[pl.ds(start, TW)], xbuf)
    pltpu.sync_copy(b_hbm.at[pl.ds(start, TW + 16)], ids)

    one = jnp.ones((16,), jnp.float32)

    def _row(r, _):
        g = ids[pl.ds(r, 16)][0]
        cnt[g, :] += one
        for f in range(NF16):
            v = xbuf[r, pl.ds(f * 16, 16)]
            acc[g, pl.ds(f * 16, 16)] += v
            acc[g, pl.ds(F + f * 16, 16)] += v * v
        return _
    lax.fori_loop(0, TW, _row, 0)

    pltpu.sync_copy(acc, psum_hbm.at[wid])
    pltpu.sync_copy(cnt, pcnt_hbm.at[wid])


def _sc_partials(x, batch):
    run = pl.kernel(
        _sc_stats,
        out_type=[
            jax.ShapeDtypeStruct((NW, G, 2 * F), jnp.float32),
            jax.ShapeDtypeStruct((NW, G, 16), jnp.float32),
        ],
        mesh=plsc.VectorSubcoreMesh(core_axis_name="c", subcore_axis_name="s"),
        scratch_types=[
            pltpu.VMEM((TW, F), jnp.float32),
            pltpu.VMEM((TW + 16,), jnp.int32),
            pltpu.VMEM((G, 2 * F), jnp.float32),
            pltpu.VMEM((G, 16), jnp.float32),
        ],
    )
    return run(x, batch)


def _tc_body(x_ref, b_ref, psum_ref, pcnt_ref, w_ref, bias_ref, ms_ref, y_ref,
             stash, sumsq, cnt, ab):
    p = pl.program_id(0)
    j = pl.program_id(1)
    b = b_ref[0, 0, :]
    oh = (b[:, None] == jax.lax.broadcasted_iota(jnp.int32, (BLK, G), 1)).astype(
        jnp.bfloat16
    )

    @pl.when((p == 0) & (j < HEAD_BLKS))
    def _stats():
        x = x_ref[...]
        xb = x.astype(jnp.bfloat16)
        stash[pl.ds(j * BLK, BLK), :] = xb
        head_rows = (j * BLK + jax.lax.broadcasted_iota(jnp.int32, (BLK, G), 0)
                     ) < HEAD
        ohm = jnp.where(head_rows, oh, jnp.bfloat16(0))
        xq = jnp.concatenate([xb, xb * xb], axis=1)
        s = jax.lax.dot_general(ohm, xq, (((0,), (0,)), ((), ())),
                                preferred_element_type=jnp.float32)
        ones8 = jnp.ones((8, BLK), dtype=jnp.bfloat16)
        c = jax.lax.dot_general(ones8, ohm, (((1,), (0,)), ((), ())),
                                preferred_element_type=jnp.float32)

        @pl.when(j == 0)
        def _init():
            sumsq[...] = s
            cnt[...] = c

        @pl.when(j != 0)
        def _acc():
            sumsq[...] += s
            cnt[...] += c

    @pl.when(p == 1)
    def _apply():
        @pl.when(j == 0)
        def _tables():
            sc_sum = jnp.sum(psum_ref[...], axis=0)
            sc_cnt = jnp.sum(pcnt_ref[...], axis=0)[:, 0]
            tot = sumsq[...] + sc_sum
            c_all = jnp.maximum(cnt[0, :] + sc_cnt, 1.0)[:, None]
            inv_c = 1.0 / c_all
            m = tot[:, :F] * inv_c
            qm = tot[:, F:] * inv_c
            ms = ms_ref[0, :][None, :]
            var = qm - m * m * ms * (2.0 - ms)
            a = w_ref[0, :][None, :] * jax.lax.rsqrt(var + EPS)
            ab[:, :F] = a.astype(jnp.bfloat16)
            ab[:, F:] = (bias_ref[0, :][None, :] - a * m * ms).astype(jnp.bfloat16)

        abrow = jax.lax.dot_general(oh, ab[...], (((1,), (0,)), ((), ())),
                                    preferred_element_type=jnp.float32)

        @pl.when(j < HEAD_BLKS)
        def _from_stash():
            xs = stash[pl.ds(j * BLK, BLK), :].astype(jnp.float32)
            y_ref[...] = abrow[:, :F] * xs + abrow[:, F:]

        @pl.when(j >= HEAD_BLKS)
        def _from_hbm():
            y_ref[...] = abrow[:, :F] * x_ref[...] + abrow[:, F:]


def _x_index(p, j):
    return (jnp.where(p == 0, jnp.minimum(j, HEAD_BLKS - 1),
                      jnp.where(j < HEAD_BLKS, HEAD_BLKS - 1, j)), 0)


@jax.jit
def kernel(x, batch, weight, bias, mean_scale):
    b32 = batch.astype(jnp.int32)
    psum, pcnt = _sc_partials(x, jnp.pad(b32, (0, 32)))
    b3 = b32.reshape(NBLK, 1, BLK)
    w2 = weight.reshape(1, F)
    bias2 = bias.reshape(1, F)
    ms2 = mean_scale.reshape(1, F)

    return pl.pallas_call(
        _tc_body,
        grid=(2, NBLK),
        in_specs=[
            pl.BlockSpec((BLK, F), _x_index),
            pl.BlockSpec((1, 1, BLK), lambda p, j: (j, 0, 0)),
            pl.BlockSpec((NW, G, 2 * F), lambda p, j: (0, 0, 0)),
            pl.BlockSpec((NW, G, 16), lambda p, j: (0, 0, 0)),
            pl.BlockSpec((1, F), lambda p, j: (0, 0)),
            pl.BlockSpec((1, F), lambda p, j: (0, 0)),
            pl.BlockSpec((1, F), lambda p, j: (0, 0)),
        ],
        out_specs=pl.BlockSpec((BLK, F), lambda p, j: (jnp.where(p == 0, 0, j), 0)),
        out_shape=jax.ShapeDtypeStruct((N, F), jnp.float32),
        scratch_shapes=[
            pltpu.VMEM((HEAD_BLKS * BLK, F), jnp.bfloat16),
            pltpu.VMEM((G, 2 * F), jnp.float32),
            pltpu.VMEM((8, G), jnp.float32),
            pltpu.VMEM((G, 2 * F), jnp.bfloat16),
        ],
    )(x, b3, psum, pcnt, w2, bias2, ms2)


# speedup vs baseline: 1.1674x; 1.1674x over previous
import jax
import jax.numpy as jnp
from jax import lax
from jax.experimental import pallas as pl
from jax.experimental.pallas import tpu as pltpu
from jax.experimental.pallas import tpu_sc as plsc

N = 100000
F = 128
G = 64
EPS = 1e-05
BLK = 10000
NBLK = N // BLK
HEAD_BLKS = 8
NW = 32
TW = 640
TAIL = NW * TW
HEAD = N - TAIL
NF16 = F // 16


def _sc_stats(x_hbm, b_hbm, psum_hbm, pcnt_hbm, xbuf, ids, acc, cnt):
    wid = lax.axis_index("s") * 2 + lax.axis_index("c")
    start = HEAD + wid * TW
    def _zrow(g, _):
        for f in range(2 * NF16):
            acc[g, pl.ds(f * 16, 16)] = jnp.zeros((16,), jnp.float32)
        cnt[g, :] = jnp.zeros((16,), jnp.float32)
        return _
    lax.fori_loop(0, G, _zrow, 0)

    pltpu.sync_copy(x_hbm.at[pl.ds(start, TW)], xbuf)
    pltpu.sync_copy(b_hbm.at[pl.ds(start, TW + 16)], ids)

    zero = jnp.zeros((16,), jnp.float32)
    zeros16 = (zero,) * (2 * NF16)

    def _flush(g, c, regs):
        cnt[g, :] += jnp.full((16,), c, jnp.float32)
        for f in range(NF16):
            acc[g, pl.ds(f * 16, 16)] += regs[f]
            acc[g, pl.ds(F + f * 16, 16)] += regs[NF16 + f]

    def _row(r, carry):
        cur_g, c = carry[0], carry[1]
        regs = carry[2:]
        g = ids[pl.ds(r, 16)][0]

        def _new(_):
            _flush(cur_g, c, regs)
            return (g, jnp.int32(0)) + zeros16

        def _same(_):
            return carry

        nc = lax.cond(g != cur_g, _new, _same, 0)
        cur_g, c, regs = nc[0], nc[1], nc[2:]
        sums_out, sqs_out = [], []
        for f in range(NF16):
            v = xbuf[r, pl.ds(f * 16, 16)]
            sums_out.append(regs[f] + v)
            sqs_out.append(regs[NF16 + f] + v * v)
        return tuple([cur_g, c + jnp.int32(1)] + sums_out + sqs_out)

    first_g = ids[pl.ds(0, 16)][0]
    final = lax.fori_loop(0, TW, _row, (first_g, jnp.int32(0)) + zeros16)
    _flush(final[0], final[1], final[2:])

    pltpu.sync_copy(acc, psum_hbm.at[wid])
    pltpu.sync_copy(cnt, pcnt_hbm.at[wid])


def _sc_partials(x, batch):
    run = pl.kernel(
        _sc_stats,
        out_type=[
            jax.ShapeDtypeStruct((NW, G, 2 * F), jnp.float32),
            jax.ShapeDtypeStruct((NW, G, 16), jnp.float32),
        ],
        mesh=plsc.VectorSubcoreMesh(core_axis_name="c", subcore_axis_name="s"),
        scratch_types=[
            pltpu.VMEM((TW, F), jnp.float32),
            pltpu.VMEM((TW + 16,), jnp.int32),
            pltpu.VMEM((G, 2 * F), jnp.float32),
            pltpu.VMEM((G, 16), jnp.float32),
        ],
    )
    return run(x, batch)


def _tc_body(x_ref, b_ref, psum_ref, pcnt_ref, w_ref, bias_ref, ms_ref, y_ref,
             stash, sumsq, cnt, ab):
    p = pl.program_id(0)
    j = pl.program_id(1)
    b = b_ref[0, 0, :]
    oh = (b[:, None] == jax.lax.broadcasted_iota(jnp.int32, (BLK, G), 1)).astype(
        jnp.bfloat16
    )

    @pl.when((p == 0) & (j < HEAD_BLKS))
    def _stats():
        x = x_ref[...]
        xb = x.astype(jnp.bfloat16)
        stash[pl.ds(j * BLK, BLK), :] = xb
        head_rows = (j * BLK + jax.lax.broadcasted_iota(jnp.int32, (BLK, G), 0)
                     ) < HEAD
        ohm = jnp.where(head_rows, oh, jnp.bfloat16(0))
        xq = jnp.concatenate([xb, xb * xb], axis=1)
        s = jax.lax.dot_general(ohm, xq, (((0,), (0,)), ((), ())),
                                preferred_element_type=jnp.float32)
        ones8 = jnp.ones((8, BLK), dtype=jnp.bfloat16)
        c = jax.lax.dot_general(ones8, ohm, (((1,), (0,)), ((), ())),
                                preferred_element_type=jnp.float32)

        @pl.when(j == 0)
        def _init():
            sumsq[...] = s
            cnt[...] = c

        @pl.when(j != 0)
        def _acc():
            sumsq[...] += s
            cnt[...] += c

    @pl.when(p == 1)
    def _apply():
        @pl.when(j == 0)
        def _tables():
            sc_sum = jnp.sum(psum_ref[...], axis=0)
            sc_cnt = jnp.sum(pcnt_ref[...], axis=0)[:, 0]
            tot = sumsq[...] + sc_sum
            c_all = jnp.maximum(cnt[0, :] + sc_cnt, 1.0)[:, None]
            inv_c = 1.0 / c_all
            m = tot[:, :F] * inv_c
            qm = tot[:, F:] * inv_c
            ms = ms_ref[0, :][None, :]
            var = qm - m * m * ms * (2.0 - ms)
            a = w_ref[0, :][None, :] * jax.lax.rsqrt(var + EPS)
            ab[:, :F] = a.astype(jnp.bfloat16)
            ab[:, F:] = (bias_ref[0, :][None, :] - a * m * ms).astype(jnp.bfloat16)

        abrow = jax.lax.dot_general(oh, ab[...], (((1,), (0,)), ((), ())),
                                    preferred_element_type=jnp.float32)

        @pl.when(j < HEAD_BLKS)
        def _from_stash():
            xs = stash[pl.ds(j * BLK, BLK), :].astype(jnp.float32)
            y_ref[...] = abrow[:, :F] * xs + abrow[:, F:]

        @pl.when(j >= HEAD_BLKS)
        def _from_hbm():
            y_ref[...] = abrow[:, :F] * x_ref[...] + abrow[:, F:]


def _x_index(p, j):
    return (jnp.where(p == 0, jnp.minimum(j, HEAD_BLKS - 1),
                      jnp.where(j < HEAD_BLKS, HEAD_BLKS - 1, j)), 0)


@jax.jit
def kernel(x, batch, weight, bias, mean_scale):
    b32 = batch.astype(jnp.int32)
    psum, pcnt = _sc_partials(x, jnp.pad(b32, (0, 32)))
    b3 = b32.reshape(NBLK, 1, BLK)
    w2 = weight.reshape(1, F)
    bias2 = bias.reshape(1, F)
    ms2 = mean_scale.reshape(1, F)

    return pl.pallas_call(
        _tc_body,
        grid=(2, NBLK),
        in_specs=[
            pl.BlockSpec((BLK, F), _x_index),
            pl.BlockSpec((1, 1, BLK), lambda p, j: (j, 0, 0)),
            pl.BlockSpec((NW, G, 2 * F), lambda p, j: (0, 0, 0)),
            pl.BlockSpec((NW, G, 16), lambda p, j: (0, 0, 0)),
            pl.BlockSpec((1, F), lambda p, j: (0, 0)),
            pl.BlockSpec((1, F), lambda p, j: (0, 0)),
            pl.BlockSpec((1, F), lambda p, j: (0, 0)),
        ],
        out_specs=pl.BlockSpec((BLK, F), lambda p, j: (jnp.where(p == 0, 0, j), 0)),
        out_shape=jax.ShapeDtypeStruct((N, F), jnp.float32),
        scratch_shapes=[
            pltpu.VMEM((HEAD_BLKS * BLK, F), jnp.bfloat16),
            pltpu.VMEM((G, 2 * F), jnp.float32),
            pltpu.VMEM((8, G), jnp.float32),
            pltpu.VMEM((G, 2 * F), jnp.bfloat16),
        ],
    )(x, b3, psum, pcnt, w2, bias2, ms2)


# speedup vs baseline: 1.2578x; 1.0775x over previous
import jax
import jax.numpy as jnp
from jax import lax
from jax.experimental import pallas as pl
from jax.experimental.pallas import tpu as pltpu
from jax.experimental.pallas import tpu_sc as plsc

N = 100000
F = 128
G = 64
EPS = 1e-05
BLK = 10000
NBLK = N // BLK
HEAD_BLKS = 8
NW = 32
TW = 640
TAIL = NW * TW
HEAD = N - TAIL
NF16 = F // 16


def _sc_stats(x_hbm, b_hbm, psum_hbm, pcnt_hbm, xbuf, ids, acc, cnt, run):
    wid = lax.axis_index("s") * 2 + lax.axis_index("c")
    start = HEAD + wid * TW
    def _zrow(g, _):
        for f in range(2 * NF16):
            acc[g, pl.ds(f * 16, 16)] = jnp.zeros((16,), jnp.float32)
        cnt[g, :] = jnp.zeros((16,), jnp.float32)
        return _
    lax.fori_loop(0, G, _zrow, 0)

    pltpu.sync_copy(x_hbm.at[pl.ds(start, TW)], xbuf)
    pltpu.sync_copy(b_hbm.at[pl.ds(start, TW + 16)], ids)

    zero = jnp.zeros((16,), jnp.float32)
    one = jnp.ones((16,), jnp.float32)

    def _flush(g, c):
        cnt[g, :] += jnp.full((16,), c, jnp.float32)
        for f in range(2 * NF16):
            acc[g, pl.ds(f * 16, 16)] += run[pl.ds(f * 16, 16)]
            run[pl.ds(f * 16, 16)] = zero

    def _group(gi, carry):
        r0 = gi * 16
        idvec = ids[pl.ds(r0, 16)]
        g0 = idvec[0]
        g15 = idvec[15]
        cur_g, c = carry

        def _boundary(_):
            _flush(cur_g, c)
            return (g0, jnp.int32(0))

        def _same(_):
            return carry

        cur_g, c = lax.cond(g0 != cur_g, _boundary, _same, 0)

        def _uniform(_):
            for f in range(NF16):
                s = run[pl.ds(f * 16, 16)]
                q = run[pl.ds(F + f * 16, 16)]
                for r in range(16):
                    v = xbuf[r0 + r, pl.ds(f * 16, 16)]
                    s = s + v
                    q = q + v * v
                run[pl.ds(f * 16, 16)] = s
                run[pl.ds(F + f * 16, 16)] = q
            return (g0, c + jnp.int32(16))

        def _mixed(_):
            _flush(cur_g, c)
            for r in range(16):
                g = idvec[r]
                cnt[g, :] += one
                for f in range(NF16):
                    v = xbuf[r0 + r, pl.ds(f * 16, 16)]
                    acc[g, pl.ds(f * 16, 16)] += v
                    acc[g, pl.ds(F + f * 16, 16)] += v * v
            return (g15, jnp.int32(0))

        return lax.cond(g0 == g15, _uniform, _mixed, 0)

    def _zrun(f, _):
        run[pl.ds(f * 16, 16)] = zero
        return _
    lax.fori_loop(0, 2 * NF16, _zrun, 0)
    first_g = ids[pl.ds(0, 16)][0]
    final = lax.fori_loop(0, TW // 16, _group, (first_g, jnp.int32(0)))
    _flush(final[0], final[1])

    pltpu.sync_copy(acc, psum_hbm.at[wid])
    pltpu.sync_copy(cnt, pcnt_hbm.at[wid])


def _sc_partials(x, batch):
    run = pl.kernel(
        _sc_stats,
        out_type=[
            jax.ShapeDtypeStruct((NW, G, 2 * F), jnp.float32),
            jax.ShapeDtypeStruct((NW, G, 16), jnp.float32),
        ],
        mesh=plsc.VectorSubcoreMesh(core_axis_name="c", subcore_axis_name="s"),
        scratch_types=[
            pltpu.VMEM((TW, F), jnp.float32),
            pltpu.VMEM((TW + 16,), jnp.int32),
            pltpu.VMEM((G, 2 * F), jnp.float32),
            pltpu.VMEM((G, 16), jnp.float32),
            pltpu.VMEM((2 * F,), jnp.float32),
        ],
    )
    return run(x, batch)


def _tc_body(x_ref, b_ref, psum_ref, pcnt_ref, w_ref, bias_ref, ms_ref, y_ref,
             stash, sumsq, cnt, ab):
    p = pl.program_id(0)
    j = pl.program_id(1)
    b = b_ref[0, 0, :]
    oh = (b[:, None] == jax.lax.broadcasted_iota(jnp.int32, (BLK, G), 1)).astype(
        jnp.bfloat16
    )

    @pl.when((p == 0) & (j < HEAD_BLKS))
    def _stats():
        x = x_ref[...]
        xb = x.astype(jnp.bfloat16)
        stash[pl.ds(j * BLK, BLK), :] = xb
        head_rows = (j * BLK + jax.lax.broadcasted_iota(jnp.int32, (BLK, G), 0)
                     ) < HEAD
        ohm = jnp.where(head_rows, oh, jnp.bfloat16(0))
        xq = jnp.concatenate([xb, xb * xb], axis=1)
        s = jax.lax.dot_general(ohm, xq, (((0,), (0,)), ((), ())),
                                preferred_element_type=jnp.float32)
        ones8 = jnp.ones((8, BLK), dtype=jnp.bfloat16)
        c = jax.lax.dot_general(ones8, ohm, (((1,), (0,)), ((), ())),
                                preferred_element_type=jnp.float32)

        @pl.when(j == 0)
        def _init():
            sumsq[...] = s
            cnt[...] = c

        @pl.when(j != 0)
        def _acc():
            sumsq[...] += s
            cnt[...] += c

    @pl.when(p == 1)
    def _apply():
        @pl.when(j == 0)
        def _tables():
            sc_sum = jnp.sum(psum_ref[...], axis=0)
            sc_cnt = jnp.sum(pcnt_ref[...], axis=0)[:, 0]
            tot = sumsq[...] + sc_sum
            c_all = jnp.maximum(cnt[0, :] + sc_cnt, 1.0)[:, None]
            inv_c = 1.0 / c_all
            m = tot[:, :F] * inv_c
            qm = tot[:, F:] * inv_c
            ms = ms_ref[0, :][None, :]
            var = qm - m * m * ms * (2.0 - ms)
            a = w_ref[0, :][None, :] * jax.lax.rsqrt(var + EPS)
            ab[:, :F] = a.astype(jnp.bfloat16)
            ab[:, F:] = (bias_ref[0, :][None, :] - a * m * ms).astype(jnp.bfloat16)

        abrow = jax.lax.dot_general(oh, ab[...], (((1,), (0,)), ((), ())),
                                    preferred_element_type=jnp.float32)

        @pl.when(j < HEAD_BLKS)
        def _from_stash():
            xs = stash[pl.ds(j * BLK, BLK), :].astype(jnp.float32)
            y_ref[...] = abrow[:, :F] * xs + abrow[:, F:]

        @pl.when(j >= HEAD_BLKS)
        def _from_hbm():
            y_ref[...] = abrow[:, :F] * x_ref[...] + abrow[:, F:]


def _x_index(p, j):
    return (jnp.where(p == 0, jnp.minimum(j, HEAD_BLKS - 1),
                      jnp.where(j < HEAD_BLKS, HEAD_BLKS - 1, j)), 0)


@jax.jit
def kernel(x, batch, weight, bias, mean_scale):
    b32 = batch.astype(jnp.int32)
    psum, pcnt = _sc_partials(x, jnp.pad(b32, (0, 32)))
    b3 = b32.reshape(NBLK, 1, BLK)
    w2 = weight.reshape(1, F)
    bias2 = bias.reshape(1, F)
    ms2 = mean_scale.reshape(1, F)

    return pl.pallas_call(
        _tc_body,
        grid=(2, NBLK),
        in_specs=[
            pl.BlockSpec((BLK, F), _x_index),
            pl.BlockSpec((1, 1, BLK), lambda p, j: (j, 0, 0)),
            pl.BlockSpec((NW, G, 2 * F), lambda p, j: (0, 0, 0)),
            pl.BlockSpec((NW, G, 16), lambda p, j: (0, 0, 0)),
            pl.BlockSpec((1, F), lambda p, j: (0, 0)),
            pl.BlockSpec((1, F), lambda p, j: (0, 0)),
            pl.BlockSpec((1, F), lambda p, j: (0, 0)),
        ],
        out_specs=pl.BlockSpec((BLK, F), lambda p, j: (jnp.where(p == 0, 0, j), 0)),
        out_shape=jax.ShapeDtypeStruct((N, F), jnp.float32),
        scratch_shapes=[
            pltpu.VMEM((HEAD_BLKS * BLK, F), jnp.bfloat16),
            pltpu.VMEM((G, 2 * F), jnp.float32),
            pltpu.VMEM((8, G), jnp.float32),
            pltpu.VMEM((G, 2 * F), jnp.bfloat16),
        ],
    )(x, b3, psum, pcnt, w2, bias2, ms2)


# speedup vs baseline: 1.3129x; 1.0438x over previous
import jax
import jax.numpy as jnp
from jax import lax
from jax.experimental import pallas as pl
from jax.experimental.pallas import tpu as pltpu
from jax.experimental.pallas import tpu_sc as plsc

N = 100000
F = 128
G = 64
EPS = 1e-05
BLK = 10000
NBLK = N // BLK
HEAD_BLKS = 9
NW = 32
TW = 320
TAIL = NW * TW
HEAD = N - TAIL
NF16 = F // 16


def _sc_stats(x_hbm, b_hbm, psum_hbm, pcnt_hbm, xbuf, ids, acc, cnt, run):
    wid = lax.axis_index("s") * 2 + lax.axis_index("c")
    start = HEAD + wid * TW
    def _zrow(g, _):
        for f in range(2 * NF16):
            acc[g, pl.ds(f * 16, 16)] = jnp.zeros((16,), jnp.float32)
        cnt[g, :] = jnp.zeros((16,), jnp.float32)
        return _
    lax.fori_loop(0, G, _zrow, 0)

    pltpu.sync_copy(x_hbm.at[pl.ds(start, TW)], xbuf)
    pltpu.sync_copy(b_hbm.at[pl.ds(start, TW + 16)], ids)

    zero = jnp.zeros((16,), jnp.float32)
    one = jnp.ones((16,), jnp.float32)

    def _flush(g, c):
        cnt[g, :] += jnp.full((16,), c, jnp.float32)
        for f in range(2 * NF16):
            acc[g, pl.ds(f * 16, 16)] += run[pl.ds(f * 16, 16)]
            run[pl.ds(f * 16, 16)] = zero

    def _group(gi, carry):
        r0 = gi * 16
        idvec = ids[pl.ds(r0, 16)]
        g0 = idvec[0]
        g15 = idvec[15]
        cur_g, c = carry

        def _boundary(_):
            _flush(cur_g, c)
            return (g0, jnp.int32(0))

        def _same(_):
            return carry

        cur_g, c = lax.cond(g0 != cur_g, _boundary, _same, 0)

        def _uniform(_):
            for f in range(NF16):
                s = run[pl.ds(f * 16, 16)]
                q = run[pl.ds(F + f * 16, 16)]
                for r in range(16):
                    v = xbuf[r0 + r, pl.ds(f * 16, 16)]
                    s = s + v
                    q = q + v * v
                run[pl.ds(f * 16, 16)] = s
                run[pl.ds(F + f * 16, 16)] = q
            return (g0, c + jnp.int32(16))

        def _mixed(_):
            _flush(cur_g, c)
            for r in range(16):
                g = idvec[r]
                cnt[g, :] += one
                for f in range(NF16):
                    v = xbuf[r0 + r, pl.ds(f * 16, 16)]
                    acc[g, pl.ds(f * 16, 16)] += v
                    acc[g, pl.ds(F + f * 16, 16)] += v * v
            return (g15, jnp.int32(0))

        return lax.cond(g0 == g15, _uniform, _mixed, 0)

    def _zrun(f, _):
        run[pl.ds(f * 16, 16)] = zero
        return _
    lax.fori_loop(0, 2 * NF16, _zrun, 0)
    first_g = ids[pl.ds(0, 16)][0]
    final = lax.fori_loop(0, TW // 16, _group, (first_g, jnp.int32(0)))
    _flush(final[0], final[1])

    pltpu.sync_copy(acc, psum_hbm.at[wid])
    pltpu.sync_copy(cnt, pcnt_hbm.at[wid])


def _sc_partials(x, batch):
    run = pl.kernel(
        _sc_stats,
        out_type=[
            jax.ShapeDtypeStruct((NW, G, 2 * F), jnp.float32),
            jax.ShapeDtypeStruct((NW, G, 16), jnp.float32),
        ],
        mesh=plsc.VectorSubcoreMesh(core_axis_name="c", subcore_axis_name="s"),
        scratch_types=[
            pltpu.VMEM((TW, F), jnp.float32),
            pltpu.VMEM((TW + 16,), jnp.int32),
            pltpu.VMEM((G, 2 * F), jnp.float32),
            pltpu.VMEM((G, 16), jnp.float32),
            pltpu.VMEM((2 * F,), jnp.float32),
        ],
    )
    return run(x, batch)


def _tc_body(x_ref, b_ref, psum_ref, pcnt_ref, w_ref, bias_ref, ms_ref, y_ref,
             stash, sumsq, cnt, ab):
    p = pl.program_id(0)
    j = pl.program_id(1)
    b = b_ref[0, 0, :]
    oh = (b[:, None] == jax.lax.broadcasted_iota(jnp.int32, (BLK, G), 1)).astype(
        jnp.bfloat16
    )

    @pl.when((p == 0) & (j < HEAD_BLKS))
    def _stats():
        x = x_ref[...]
        xb = x.astype(jnp.bfloat16)
        stash[pl.ds(j * BLK, BLK), :] = xb
        head_rows = (j * BLK + jax.lax.broadcasted_iota(jnp.int32, (BLK, G), 0)
                     ) < HEAD
        ohm = jnp.where(head_rows, oh, jnp.bfloat16(0))
        xq = jnp.concatenate([xb, xb * xb], axis=1)
        s = jax.lax.dot_general(ohm, xq, (((0,), (0,)), ((), ())),
                                preferred_element_type=jnp.float32)
        ones8 = jnp.ones((8, BLK), dtype=jnp.bfloat16)
        c = jax.lax.dot_general(ones8, ohm, (((1,), (0,)), ((), ())),
                                preferred_element_type=jnp.float32)

        @pl.when(j == 0)
        def _init():
            sumsq[...] = s
            cnt[...] = c

        @pl.when(j != 0)
        def _acc():
            sumsq[...] += s
            cnt[...] += c

    @pl.when(p == 1)
    def _apply():
        @pl.when(j == 0)
        def _tables():
            sc_sum = jnp.sum(psum_ref[...], axis=0)
            sc_cnt = jnp.sum(pcnt_ref[...], axis=0)[:, 0]
            tot = sumsq[...] + sc_sum
            c_all = jnp.maximum(cnt[0, :] + sc_cnt, 1.0)[:, None]
            inv_c = 1.0 / c_all
            m = tot[:, :F] * inv_c
            qm = tot[:, F:] * inv_c
            ms = ms_ref[0, :][None, :]
            var = qm - m * m * ms * (2.0 - ms)
            a = w_ref[0, :][None, :] * jax.lax.rsqrt(var + EPS)
            ab[:, :F] = a.astype(jnp.bfloat16)
            ab[:, F:] = (bias_ref[0, :][None, :] - a * m * ms).astype(jnp.bfloat16)

        abrow = jax.lax.dot_general(oh, ab[...], (((1,), (0,)), ((), ())),
                                    preferred_element_type=jnp.float32)

        @pl.when(j < HEAD_BLKS)
        def _from_stash():
            xs = stash[pl.ds(j * BLK, BLK), :].astype(jnp.float32)
            y_ref[...] = abrow[:, :F] * xs + abrow[:, F:]

        @pl.when(j >= HEAD_BLKS)
        def _from_hbm():
            y_ref[...] = abrow[:, :F] * x_ref[...] + abrow[:, F:]


def _x_index(p, j):
    return (jnp.where(p == 0, jnp.minimum(j, HEAD_BLKS - 1),
                      jnp.where(j < HEAD_BLKS, HEAD_BLKS - 1, j)), 0)


@jax.jit
def kernel(x, batch, weight, bias, mean_scale):
    b32 = batch.astype(jnp.int32)
    psum, pcnt = _sc_partials(x, jnp.pad(b32, (0, 32)))
    b3 = b32.reshape(NBLK, 1, BLK)
    w2 = weight.reshape(1, F)
    bias2 = bias.reshape(1, F)
    ms2 = mean_scale.reshape(1, F)

    return pl.pallas_call(
        _tc_body,
        grid=(2, NBLK),
        in_specs=[
            pl.BlockSpec((BLK, F), _x_index),
            pl.BlockSpec((1, 1, BLK), lambda p, j: (j, 0, 0)),
            pl.BlockSpec((NW, G, 2 * F), lambda p, j: (0, 0, 0)),
            pl.BlockSpec((NW, G, 16), lambda p, j: (0, 0, 0)),
            pl.BlockSpec((1, F), lambda p, j: (0, 0)),
            pl.BlockSpec((1, F), lambda p, j: (0, 0)),
            pl.BlockSpec((1, F), lambda p, j: (0, 0)),
        ],
        out_specs=pl.BlockSpec((BLK, F), lambda p, j: (jnp.where(p == 0, 0, j), 0)),
        out_shape=jax.ShapeDtypeStruct((N, F), jnp.float32),
        scratch_shapes=[
            pltpu.VMEM((HEAD_BLKS * BLK, F), jnp.bfloat16),
            pltpu.VMEM((G, 2 * F), jnp.float32),
            pltpu.VMEM((8, G), jnp.float32),
            pltpu.VMEM((G, 2 * F), jnp.bfloat16),
        ],
    )(x, b3, psum, pcnt, w2, bias2, ms2)
